# Initial kernel scaffold; baseline (speedup 1.0000x reference)
import jax
import jax.numpy as jnp
from jax.experimental import pallas as pl


def kernel(h, edge_index, e, pos_enc, graph_ids, Wpe, bpe, eps, W1, b1, g1, bt1, W2, b2, ga, ba, gl, bl, Wp, bp):
    raise NotImplementedError("write your pallas kernel here")


# speedup vs baseline: 2.9600x; 2.9600x over previous
import functools

import jax
import jax.numpy as jnp
from jax import lax
from jax.experimental import pallas as pl
from jax.experimental.pallas import tpu as pltpu
from jax.experimental.pallas import tpu_sc as plsc

N_NODES = 10000
N_EDGES = 160000
HID = 256
HALF = 128
PE_DIM = 20
NLAYERS = 4
NGRAPH = 64
NCLS = 10

BLK = 2000
NB = N_NODES // BLK

NSUB = 16
EDGES_PER_TILE = N_EDGES // NSUB
CHUNK = 80
NCHUNK = EDGES_PER_TILE // CHUNK
ROWS_PER_TILE = N_NODES // NSUB
ZR = 125


def _seg_body(x2_hbm, src_hbm, dst_hbm, out_hbm, sidx, didx, rows, zbuf, acc, sem):
    c = lax.axis_index("c")
    s = lax.axis_index("s")
    zero16 = jnp.zeros((16,), jnp.float32)

    @pl.loop(0, ZR)
    def _zero_row(r):
        for v in range(HALF // 16):
            zbuf[r, pl.ds(v * 16, 16)] = zero16

    @pl.loop(0, ROWS_PER_TILE // ZR)
    def _zero_acc(kk):
        pltpu.sync_copy(zbuf, acc.at[pl.ds(s * ROWS_PER_TILE + kk * ZR, ZR)])

    plsc.subcore_barrier()

    off = c * N_NODES

    @pl.loop(0, NCHUNK)
    def _edges(j):
        base = s * EDGES_PER_TILE + j * CHUNK
        pltpu.sync_copy(src_hbm.at[pl.ds(base, CHUNK)], sidx)
        pltpu.sync_copy(dst_hbm.at[pl.ds(base, CHUNK)], didx)
        for v in range(CHUNK // 16):
            sidx[pl.ds(v * 16, 16)] = sidx[pl.ds(v * 16, 16)] + off
        pltpu.async_copy(x2_hbm.at[sidx], rows, sem).wait()
        pltpu.sync_copy(rows, acc.at[didx], add=True)

    plsc.subcore_barrier()
    pltpu.sync_copy(
        acc.at[pl.ds(s * ROWS_PER_TILE, ROWS_PER_TILE)],
        out_hbm.at[pl.ds(off + s * ROWS_PER_TILE, ROWS_PER_TILE)],
    )


def _segment_sum_sc(x2, src, dst):
    mesh = plsc.VectorSubcoreMesh(core_axis_name="c", subcore_axis_name="s")
    fn = pl.kernel(
        _seg_body,
        out_type=jax.ShapeDtypeStruct((2 * N_NODES, HALF), jnp.float32),
        mesh=mesh,
        scratch_types=[
            pltpu.VMEM((CHUNK,), jnp.int32),
            pltpu.VMEM((CHUNK,), jnp.int32),
            pltpu.VMEM((CHUNK, HALF), jnp.float32),
            pltpu.VMEM((ZR, HALF), jnp.float32),
            pltpu.VMEM_SHARED((N_NODES, HALF), jnp.float32),
            pltpu.SemaphoreType.DMA,
        ],
        compiler_params=pltpu.CompilerParams(use_tc_tiling_on_sc=False),
    )
    return fn(x2, src, dst)


_INV_N = 1.0 / N_NODES


def _bn_coeffs(st_sum, st_sq, g, b):
    mu = st_sum * _INV_N
    var = st_sq * _INV_N - mu * mu
    sc = g * lax.rsqrt(var + 1e-5)
    sh = b - mu * sc
    return sc, sh


def _accum_stats(st_ref, z, i):
    @pl.when(i == 0)
    def _():
        st_ref[...] = jnp.zeros_like(st_ref)

    s1 = jnp.sum(z, axis=0)
    s2 = jnp.sum(z * z, axis=0)
    st_ref[...] += jnp.concatenate([s1[None, None, :], s2[None, None, :]], axis=1)


def _proj_body(p_ref, w_ref, b_ref, o_ref):
    o_ref[...] = (
        jnp.dot(p_ref[...], w_ref[0], preferred_element_type=jnp.float32)
        + b_ref[0, 0][None, :]
    )


def _proj(pos_enc, wq, bq):
    return pl.pallas_call(
        _proj_body,
        grid=(2, NB),
        in_specs=[
            pl.BlockSpec((BLK, PE_DIM), lambda h, i: (i, 0)),
            pl.BlockSpec((1, PE_DIM, HALF), lambda h, i: (h, 0, 0)),
            pl.BlockSpec((1, 1, HALF), lambda h, i: (h, 0, 0)),
        ],
        out_specs=pl.BlockSpec((BLK, HALF), lambda h, i: (h * NB + i, 0)),
        out_shape=jax.ShapeDtypeStruct((2 * N_NODES, HALF), jnp.float32),
    )(pos_enc, wq, bq)


def _mlp1_body(e_ref, xlo, xhi, nlo, nhi, w_ref, b_ref, z_ref, st_ref):
    i = pl.program_id(1)
    efac = 1.0 + e_ref[0, 0]
    y = jnp.concatenate(
        [efac * xlo[...] + nlo[...], efac * xhi[...] + nhi[...]], axis=1
    )
    z = jnp.dot(y, w_ref[0], preferred_element_type=jnp.float32) + b_ref[0, 0][None, :]
    z_ref[...] = z
    _accum_stats(st_ref, z, i)


def _mlp1(eps_i, x2, n2, wq, bq):
    return pl.pallas_call(
        _mlp1_body,
        grid=(2, NB),
        in_specs=[
            pl.BlockSpec((1, 1), lambda h, i: (0, 0)),
            pl.BlockSpec((BLK, HALF), lambda h, i: (i, 0)),
            pl.BlockSpec((BLK, HALF), lambda h, i: (NB + i, 0)),
            pl.BlockSpec((BLK, HALF), lambda h, i: (i, 0)),
            pl.BlockSpec((BLK, HALF), lambda h, i: (NB + i, 0)),
            pl.BlockSpec((1, HID, HALF), lambda h, i: (h, 0, 0)),
            pl.BlockSpec((1, 1, HALF), lambda h, i: (h, 0, 0)),
        ],
        out_specs=[
            pl.BlockSpec((BLK, HALF), lambda h, i: (h * NB + i, 0)),
            pl.BlockSpec((1, 2, HALF), lambda h, i: (h, 0, 0)),
        ],
        out_shape=[
            jax.ShapeDtypeStruct((2 * N_NODES, HALF), jnp.float32),
            jax.ShapeDtypeStruct((2, 2, HALF), jnp.float32),
        ],
    )(eps_i, x2, x2, n2, n2, wq, bq)


def _mlp2_body(st1, g_ref, bt_ref, w_ref, b_ref, zlo, zhi, z_ref, st_ref):
    i = pl.program_id(1)
    parts = []
    for a, zr in ((0, zlo), (1, zhi)):
        sc, sh = _bn_coeffs(st1[a, 0, :], st1[a, 1, :], g_ref[a], bt_ref[a])
        parts.append(jnp.maximum(zr[...] * sc[None, :] + sh[None, :], 0.0))
    y = jnp.concatenate(parts, axis=1)
    z = jnp.dot(y, w_ref[0], preferred_element_type=jnp.float32) + b_ref[0, 0][None, :]
    z_ref[...] = z
    _accum_stats(st_ref, z, i)


def _mlp2(st1, g, bt, z1, wq, bq):
    return pl.pallas_call(
        _mlp2_body,
        grid=(2, NB),
        in_specs=[
            pl.BlockSpec((2, 2, HALF), lambda h, i: (0, 0, 0)),
            pl.BlockSpec((2, HALF), lambda h, i: (0, 0)),
            pl.BlockSpec((2, HALF), lambda h, i: (0, 0)),
            pl.BlockSpec((1, HID, HALF), lambda h, i: (h, 0, 0)),
            pl.BlockSpec((1, 1, HALF), lambda h, i: (h, 0, 0)),
            pl.BlockSpec((BLK, HALF), lambda h, i: (i, 0)),
            pl.BlockSpec((BLK, HALF), lambda h, i: (NB + i, 0)),
        ],
        out_specs=[
            pl.BlockSpec((BLK, HALF), lambda h, i: (h * NB + i, 0)),
            pl.BlockSpec((1, 2, HALF), lambda h, i: (h, 0, 0)),
        ],
        out_shape=[
            jax.ShapeDtypeStruct((2 * N_NODES, HALF), jnp.float32),
            jax.ShapeDtypeStruct((2, 2, HALF), jnp.float32),
        ],
    )(st1, g, bt, wq, bq, z1, z1)


def _bnrelu_body(st_in, g_ref, b_ref, z_ref, o_ref, st_ref):
    i = pl.program_id(1)
    sc, sh = _bn_coeffs(st_in[0, 0, :], st_in[0, 1, :], g_ref[0, 0], b_ref[0, 0])
    val = jnp.maximum(z_ref[...] * sc[None, :] + sh[None, :], 0.0)
    o_ref[...] = val
    _accum_stats(st_ref, val, i)


def _bnrelu(st_in, g, b, z):
    return pl.pallas_call(
        _bnrelu_body,
        grid=(2, NB),
        in_specs=[
            pl.BlockSpec((1, 2, HALF), lambda h, i: (h, 0, 0)),
            pl.BlockSpec((1, 1, HALF), lambda h, i: (h, 0, 0)),
            pl.BlockSpec((1, 1, HALF), lambda h, i: (h, 0, 0)),
            pl.BlockSpec((BLK, HALF), lambda h, i: (h * NB + i, 0)),
        ],
        out_specs=[
            pl.BlockSpec((BLK, HALF), lambda h, i: (h * NB + i, 0)),
            pl.BlockSpec((1, 2, HALF), lambda h, i: (h, 0, 0)),
        ],
        out_shape=[
            jax.ShapeDtypeStruct((2 * N_NODES, HALF), jnp.float32),
            jax.ShapeDtypeStruct((2, 2, HALF), jnp.float32),
        ],
    )(st_in, g, b, z)


def _bnres_body(st_in, g_ref, b_ref, bf_ref, x_ref, o_ref):
    sc, sh = _bn_coeffs(st_in[0, 0, :], st_in[0, 1, :], g_ref[0, 0], b_ref[0, 0])
    o_ref[...] = x_ref[...] + jnp.maximum(bf_ref[...] * sc[None, :] + sh[None, :], 0.0)


def _bnres(st_in, g, b, bf, x2):
    return pl.pallas_call(
        _bnres_body,
        grid=(2, NB),
        in_specs=[
            pl.BlockSpec((1, 2, HALF), lambda h, i: (h, 0, 0)),
            pl.BlockSpec((1, 1, HALF), lambda h, i: (h, 0, 0)),
            pl.BlockSpec((1, 1, HALF), lambda h, i: (h, 0, 0)),
            pl.BlockSpec((BLK, HALF), lambda h, i: (h * NB + i, 0)),
            pl.BlockSpec((BLK, HALF), lambda h, i: (h * NB + i, 0)),
        ],
        out_specs=pl.BlockSpec((BLK, HALF), lambda h, i: (h * NB + i, 0)),
        out_shape=jax.ShapeDtypeStruct((2 * N_NODES, HALF), jnp.float32),
    )(st_in, g, b, bf, x2)


def _pool_body(gid_ref, wp_ref, bp_ref, *refs):
    o_ref = refs[-1]
    h_refs = refs[:-1]
    i = pl.program_id(0)
    v = jnp.zeros((BLK, HALF), jnp.float32)
    for k in range(NLAYERS + 1):
        v = v + jnp.dot(
            h_refs[2 * k][...], wp_ref[k, 0], preferred_element_type=jnp.float32
        )
        v = v + jnp.dot(
            h_refs[2 * k + 1][...], wp_ref[k, 1], preferred_element_type=jnp.float32
        )
    gid = gid_ref[0, 0, :]
    onehot = (
        lax.broadcasted_iota(jnp.int32, (NGRAPH, BLK), 0) == gid[None, :]
    ).astype(jnp.float32)
    contrib = jnp.dot(onehot, v, preferred_element_type=jnp.float32)

    @pl.when(i == 0)
    def _():
        o_ref[...] = jnp.broadcast_to(bp_ref[0][None, :], (NGRAPH, HALF))

    o_ref[...] += contrib


def _pool(gids3, wp_all, bp_pad, hiddens):
    n_h = NLAYERS + 1
    in_specs = [
        pl.BlockSpec((1, 1, BLK), lambda i: (i, 0, 0)),
        pl.BlockSpec((n_h, 2, HALF, HALF), lambda i: (0, 0, 0, 0)),
        pl.BlockSpec((1, HALF), lambda i: (0, 0)),
    ]
    args = [gids3, wp_all, bp_pad]
    for x2 in hiddens:
        in_specs.append(pl.BlockSpec((BLK, HALF), lambda i: (i, 0)))
        in_specs.append(pl.BlockSpec((BLK, HALF), lambda i: (NB + i, 0)))
        args.append(x2)
        args.append(x2)
    return pl.pallas_call(
        _pool_body,
        grid=(NB,),
        in_specs=in_specs,
        out_specs=pl.BlockSpec((NGRAPH, HALF), lambda i: (0, 0)),
        out_shape=jax.ShapeDtypeStruct((NGRAPH, HALF), jnp.float32),
    )(*args)


def kernel(h, edge_index, e, pos_enc, graph_ids, Wpe, bpe, eps, W1, b1, g1, bt1,
           W2, b2, ga, ba, gl, bl, Wp, bp):
    src = edge_index[0]
    dst = edge_index[1]

    wpe_q = Wpe.reshape(PE_DIM, 2, HALF).transpose(1, 0, 2)
    bpe_q = bpe.reshape(2, 1, HALF)
    x2 = _proj(pos_enc, wpe_q, bpe_q)

    hiddens = [x2]
    for i in range(NLAYERS):
        w1q = W1[i].reshape(HID, 2, HALF).transpose(1, 0, 2)
        b1q = b1[i].reshape(2, 1, HALF)
        w2q = W2[i].reshape(HID, 2, HALF).transpose(1, 0, 2)
        b2q = b2[i].reshape(2, 1, HALF)
        eps_i = eps[i].reshape(1, 1)

        neigh2 = _segment_sum_sc(x2, src, dst)
        z1, st1 = _mlp1(eps_i, x2, neigh2, w1q, b1q)
        z2, st2 = _mlp2(st1, g1[i].reshape(2, HALF), bt1[i].reshape(2, HALF),
                        z1, w2q, b2q)
        bf, st3 = _bnrelu(st2, ga[i].reshape(2, 1, HALF), ba[i].reshape(2, 1, HALF), z2)
        x2 = _bnres(st3, gl[i].reshape(2, 1, HALF), bl[i].reshape(2, 1, HALF), bf, x2)
        hiddens.append(x2)

    gids3 = graph_ids.reshape(NB, 1, BLK)
    wp_all = jnp.pad(Wp, ((0, 0), (0, 0), (0, HALF - NCLS))).reshape(
        NLAYERS + 1, 2, HALF, HALF
    )
    bp_pad = jnp.pad(jnp.sum(bp, axis=0), (0, HALF - NCLS)).reshape(1, HALF)
    score_pad = _pool(gids3, wp_all, bp_pad, hiddens)
    return score_pad[:, :NCLS]


# speedup vs baseline: 5.8642x; 1.9811x over previous
import functools

import jax
import jax.numpy as jnp
from jax import lax
from jax.experimental import pallas as pl
from jax.experimental.pallas import tpu as pltpu
from jax.experimental.pallas import tpu_sc as plsc

N_NODES = 10000
N_EDGES = 160000
HID = 256
HALF = 128
PE_DIM = 20
NLAYERS = 4
NGRAPH = 64
NCLS = 10

BLK = 2000
NB = N_NODES // BLK

NSUB = 16
EDGES_PER_TILE = N_EDGES // NSUB
CHUNK = 80
NCHUNK = EDGES_PER_TILE // CHUNK
ROWS_PER_TILE = N_NODES // NSUB
ZR = 25


def _seg_body(x2_hbm, src_hbm, dst_hbm, out_hbm, sidx_all, didx_all, rows0, rows1,
              zbuf, acc, sem0, sem1):
    c = lax.axis_index("c")
    s = lax.axis_index("s")
    zero16 = jnp.zeros((16,), jnp.float32)

    pltpu.sync_copy(src_hbm.at[s], sidx_all)
    pltpu.sync_copy(dst_hbm.at[s], didx_all)
    off = c * N_NODES

    @pl.loop(0, NCHUNK)
    def _add_off(r):
        for v in range(CHUNK // 16):
            sidx_all[r, pl.ds(v * 16, 16)] = sidx_all[r, pl.ds(v * 16, 16)] + off

    @pl.loop(0, ZR)
    def _zero_row(r):
        for v in range(HALF // 16):
            zbuf[r, pl.ds(v * 16, 16)] = zero16

    @pl.loop(0, ROWS_PER_TILE // ZR)
    def _zero_acc(kk):
        pltpu.sync_copy(zbuf, acc.at[pl.ds(s * ROWS_PER_TILE + kk * ZR, ZR)])

    plsc.subcore_barrier()

    def _gather(j, buf, sem):
        pltpu.async_copy(x2_hbm.at[sidx_all.at[j]], buf, sem)

    def _wait(j, buf, sem):
        pltpu.make_async_copy(x2_hbm.at[sidx_all.at[j]], buf, sem).wait()

    def _scat(j, buf):
        pltpu.sync_copy(buf, acc.at[didx_all.at[j]], add=True)

    _gather(0, rows0, sem0)

    @pl.loop(0, (NCHUNK - 1) // 2)
    def _pipe(t):
        j2 = t * 2
        _gather(j2 + 1, rows1, sem1)
        _wait(j2, rows0, sem0)
        _scat(j2, rows0)
        _gather(j2 + 2, rows0, sem0)
        _wait(j2 + 1, rows1, sem1)
        _scat(j2 + 1, rows1)

    _wait(NCHUNK - 1, rows0, sem0)
    _scat(NCHUNK - 1, rows0)

    plsc.subcore_barrier()
    pltpu.sync_copy(
        acc.at[pl.ds(s * ROWS_PER_TILE, ROWS_PER_TILE)],
        out_hbm.at[pl.ds(off + s * ROWS_PER_TILE, ROWS_PER_TILE)],
    )


def _segment_sum_sc(x2, src3, dst3):
    mesh = plsc.VectorSubcoreMesh(core_axis_name="c", subcore_axis_name="s")
    fn = pl.kernel(
        _seg_body,
        out_type=jax.ShapeDtypeStruct((2 * N_NODES, HALF), jnp.float32),
        mesh=mesh,
        scratch_types=[
            pltpu.VMEM((NCHUNK, CHUNK), jnp.int32),
            pltpu.VMEM((NCHUNK, CHUNK), jnp.int32),
            pltpu.VMEM((CHUNK, HALF), jnp.float32),
            pltpu.VMEM((CHUNK, HALF), jnp.float32),
            pltpu.VMEM((ZR, HALF), jnp.float32),
            pltpu.VMEM_SHARED((N_NODES, HALF), jnp.float32),
            pltpu.SemaphoreType.DMA,
            pltpu.SemaphoreType.DMA,
        ],
        compiler_params=pltpu.CompilerParams(use_tc_tiling_on_sc=False),
    )
    return fn(x2, src3, dst3)


_INV_N = 1.0 / N_NODES


def _bn_coeffs(st_sum, st_sq, g, b):
    mu = st_sum * _INV_N
    var = st_sq * _INV_N - mu * mu
    sc = g * lax.rsqrt(var + 1e-5)
    sh = b - mu * sc
    return sc, sh


def _accum_stats(st_ref, z, i):
    @pl.when(i == 0)
    def _():
        st_ref[...] = jnp.zeros_like(st_ref)

    s1 = jnp.sum(z, axis=0)
    s2 = jnp.sum(z * z, axis=0)
    st_ref[...] += jnp.concatenate([s1[None, None, :], s2[None, None, :]], axis=1)


def _proj_body(p_ref, w_ref, b_ref, o_ref):
    o_ref[...] = (
        jnp.dot(p_ref[...], w_ref[0], preferred_element_type=jnp.float32)
        + b_ref[0, 0][None, :]
    )


def _proj(pos_enc, wq, bq):
    return pl.pallas_call(
        _proj_body,
        grid=(2, NB),
        in_specs=[
            pl.BlockSpec((BLK, PE_DIM), lambda h, i: (i, 0)),
            pl.BlockSpec((1, PE_DIM, HALF), lambda h, i: (h, 0, 0)),
            pl.BlockSpec((1, 1, HALF), lambda h, i: (h, 0, 0)),
        ],
        out_specs=pl.BlockSpec((BLK, HALF), lambda h, i: (h * NB + i, 0)),
        out_shape=jax.ShapeDtypeStruct((2 * N_NODES, HALF), jnp.float32),
    )(pos_enc, wq, bq)


def _mlp1_body(e_ref, xlo, xhi, nlo, nhi, w_ref, b_ref, z_ref, st_ref):
    i = pl.program_id(1)
    efac = 1.0 + e_ref[0, 0]
    y = jnp.concatenate(
        [efac * xlo[...] + nlo[...], efac * xhi[...] + nhi[...]], axis=1
    )
    z = jnp.dot(y, w_ref[0], preferred_element_type=jnp.float32) + b_ref[0, 0][None, :]
    z_ref[...] = z
    _accum_stats(st_ref, z, i)


def _mlp1(eps_i, x2, n2, wq, bq):
    return pl.pallas_call(
        _mlp1_body,
        grid=(2, NB),
        in_specs=[
            pl.BlockSpec((1, 1), lambda h, i: (0, 0)),
            pl.BlockSpec((BLK, HALF), lambda h, i: (i, 0)),
            pl.BlockSpec((BLK, HALF), lambda h, i: (NB + i, 0)),
            pl.BlockSpec((BLK, HALF), lambda h, i: (i, 0)),
            pl.BlockSpec((BLK, HALF), lambda h, i: (NB + i, 0)),
            pl.BlockSpec((1, HID, HALF), lambda h, i: (h, 0, 0)),
            pl.BlockSpec((1, 1, HALF), lambda h, i: (h, 0, 0)),
        ],
        out_specs=[
            pl.BlockSpec((BLK, HALF), lambda h, i: (h * NB + i, 0)),
            pl.BlockSpec((1, 2, HALF), lambda h, i: (h, 0, 0)),
        ],
        out_shape=[
            jax.ShapeDtypeStruct((2 * N_NODES, HALF), jnp.float32),
            jax.ShapeDtypeStruct((2, 2, HALF), jnp.float32),
        ],
    )(eps_i, x2, x2, n2, n2, wq, bq)


def _mlp2_body(st1, g_ref, bt_ref, w_ref, b_ref, zlo, zhi, z_ref, st_ref):
    i = pl.program_id(1)
    parts = []
    for a, zr in ((0, zlo), (1, zhi)):
        sc, sh = _bn_coeffs(st1[a, 0, :], st1[a, 1, :], g_ref[a], bt_ref[a])
        parts.append(jnp.maximum(zr[...] * sc[None, :] + sh[None, :], 0.0))
    y = jnp.concatenate(parts, axis=1)
    z = jnp.dot(y, w_ref[0], preferred_element_type=jnp.float32) + b_ref[0, 0][None, :]
    z_ref[...] = z
    _accum_stats(st_ref, z, i)


def _mlp2(st1, g, bt, z1, wq, bq):
    return pl.pallas_call(
        _mlp2_body,
        grid=(2, NB),
        in_specs=[
            pl.BlockSpec((2, 2, HALF), lambda h, i: (0, 0, 0)),
            pl.BlockSpec((2, HALF), lambda h, i: (0, 0)),
            pl.BlockSpec((2, HALF), lambda h, i: (0, 0)),
            pl.BlockSpec((1, HID, HALF), lambda h, i: (h, 0, 0)),
            pl.BlockSpec((1, 1, HALF), lambda h, i: (h, 0, 0)),
            pl.BlockSpec((BLK, HALF), lambda h, i: (i, 0)),
            pl.BlockSpec((BLK, HALF), lambda h, i: (NB + i, 0)),
        ],
        out_specs=[
            pl.BlockSpec((BLK, HALF), lambda h, i: (h * NB + i, 0)),
            pl.BlockSpec((1, 2, HALF), lambda h, i: (h, 0, 0)),
        ],
        out_shape=[
            jax.ShapeDtypeStruct((2 * N_NODES, HALF), jnp.float32),
            jax.ShapeDtypeStruct((2, 2, HALF), jnp.float32),
        ],
    )(st1, g, bt, wq, bq, z1, z1)


def _bnrelu_body(st_in, g_ref, b_ref, z_ref, o_ref, st_ref):
    i = pl.program_id(1)
    sc, sh = _bn_coeffs(st_in[0, 0, :], st_in[0, 1, :], g_ref[0, 0], b_ref[0, 0])
    val = jnp.maximum(z_ref[...] * sc[None, :] + sh[None, :], 0.0)
    o_ref[...] = val
    _accum_stats(st_ref, val, i)


def _bnrelu(st_in, g, b, z):
    return pl.pallas_call(
        _bnrelu_body,
        grid=(2, NB),
        in_specs=[
            pl.BlockSpec((1, 2, HALF), lambda h, i: (h, 0, 0)),
            pl.BlockSpec((1, 1, HALF), lambda h, i: (h, 0, 0)),
            pl.BlockSpec((1, 1, HALF), lambda h, i: (h, 0, 0)),
            pl.BlockSpec((BLK, HALF), lambda h, i: (h * NB + i, 0)),
        ],
        out_specs=[
            pl.BlockSpec((BLK, HALF), lambda h, i: (h * NB + i, 0)),
            pl.BlockSpec((1, 2, HALF), lambda h, i: (h, 0, 0)),
        ],
        out_shape=[
            jax.ShapeDtypeStruct((2 * N_NODES, HALF), jnp.float32),
            jax.ShapeDtypeStruct((2, 2, HALF), jnp.float32),
        ],
    )(st_in, g, b, z)


def _bnres_body(st_in, g_ref, b_ref, bf_ref, x_ref, o_ref):
    sc, sh = _bn_coeffs(st_in[0, 0, :], st_in[0, 1, :], g_ref[0, 0], b_ref[0, 0])
    o_ref[...] = x_ref[...] + jnp.maximum(bf_ref[...] * sc[None, :] + sh[None, :], 0.0)


def _bnres(st_in, g, b, bf, x2):
    return pl.pallas_call(
        _bnres_body,
        grid=(2, NB),
        in_specs=[
            pl.BlockSpec((1, 2, HALF), lambda h, i: (h, 0, 0)),
            pl.BlockSpec((1, 1, HALF), lambda h, i: (h, 0, 0)),
            pl.BlockSpec((1, 1, HALF), lambda h, i: (h, 0, 0)),
            pl.BlockSpec((BLK, HALF), lambda h, i: (h * NB + i, 0)),
            pl.BlockSpec((BLK, HALF), lambda h, i: (h * NB + i, 0)),
        ],
        out_specs=pl.BlockSpec((BLK, HALF), lambda h, i: (h * NB + i, 0)),
        out_shape=jax.ShapeDtypeStruct((2 * N_NODES, HALF), jnp.float32),
    )(st_in, g, b, bf, x2)


def _pool_body(gid_ref, wp_ref, bp_ref, *refs):
    o_ref = refs[-1]
    h_refs = refs[:-1]
    i = pl.program_id(0)
    v = jnp.zeros((BLK, HALF), jnp.float32)
    for k in range(NLAYERS + 1):
        v = v + jnp.dot(
            h_refs[2 * k][...], wp_ref[k, 0], preferred_element_type=jnp.float32
        )
        v = v + jnp.dot(
            h_refs[2 * k + 1][...], wp_ref[k, 1], preferred_element_type=jnp.float32
        )
    gid = gid_ref[0, 0, :]
    onehot = (
        lax.broadcasted_iota(jnp.int32, (NGRAPH, BLK), 0) == gid[None, :]
    ).astype(jnp.float32)
    contrib = jnp.dot(onehot, v, preferred_element_type=jnp.float32)

    @pl.when(i == 0)
    def _():
        o_ref[...] = jnp.broadcast_to(bp_ref[0][None, :], (NGRAPH, HALF))

    o_ref[...] += contrib


def _pool(gids3, wp_all, bp_pad, hiddens):
    n_h = NLAYERS + 1
    in_specs = [
        pl.BlockSpec((1, 1, BLK), lambda i: (i, 0, 0)),
        pl.BlockSpec((n_h, 2, HALF, HALF), lambda i: (0, 0, 0, 0)),
        pl.BlockSpec((1, HALF), lambda i: (0, 0)),
    ]
    args = [gids3, wp_all, bp_pad]
    for x2 in hiddens:
        in_specs.append(pl.BlockSpec((BLK, HALF), lambda i: (i, 0)))
        in_specs.append(pl.BlockSpec((BLK, HALF), lambda i: (NB + i, 0)))
        args.append(x2)
        args.append(x2)
    return pl.pallas_call(
        _pool_body,
        grid=(NB,),
        in_specs=in_specs,
        out_specs=pl.BlockSpec((NGRAPH, HALF), lambda i: (0, 0)),
        out_shape=jax.ShapeDtypeStruct((NGRAPH, HALF), jnp.float32),
    )(*args)


def kernel(h, edge_index, e, pos_enc, graph_ids, Wpe, bpe, eps, W1, b1, g1, bt1,
           W2, b2, ga, ba, gl, bl, Wp, bp):
    src3 = edge_index[0].reshape(NSUB, NCHUNK, CHUNK)
    dst3 = edge_index[1].reshape(NSUB, NCHUNK, CHUNK)

    wpe_q = Wpe.reshape(PE_DIM, 2, HALF).transpose(1, 0, 2)
    bpe_q = bpe.reshape(2, 1, HALF)
    x2 = _proj(pos_enc, wpe_q, bpe_q)

    hiddens = [x2]
    for i in range(NLAYERS):
        w1q = W1[i].reshape(HID, 2, HALF).transpose(1, 0, 2)
        b1q = b1[i].reshape(2, 1, HALF)
        w2q = W2[i].reshape(HID, 2, HALF).transpose(1, 0, 2)
        b2q = b2[i].reshape(2, 1, HALF)
        eps_i = eps[i].reshape(1, 1)

        neigh2 = _segment_sum_sc(x2, src3, dst3)
        z1, st1 = _mlp1(eps_i, x2, neigh2, w1q, b1q)
        z2, st2 = _mlp2(st1, g1[i].reshape(2, HALF), bt1[i].reshape(2, HALF),
                        z1, w2q, b2q)
        bf, st3 = _bnrelu(st2, ga[i].reshape(2, 1, HALF), ba[i].reshape(2, 1, HALF), z2)
        x2 = _bnres(st3, gl[i].reshape(2, 1, HALF), bl[i].reshape(2, 1, HALF), bf, x2)
        hiddens.append(x2)

    gids3 = graph_ids.reshape(NB, 1, BLK)
    wp_all = jnp.pad(Wp, ((0, 0), (0, 0), (0, HALF - NCLS))).reshape(
        NLAYERS + 1, 2, HALF, HALF
    )
    bp_pad = jnp.pad(jnp.sum(bp, axis=0), (0, HALF - NCLS)).reshape(1, HALF)
    score_pad = _pool(gids3, wp_all, bp_pad, hiddens)
    return score_pad[:, :NCLS]


# speedup vs baseline: 6.4389x; 1.0980x over previous
import functools

import jax
import jax.numpy as jnp
from jax import lax
from jax.experimental import pallas as pl
from jax.experimental.pallas import tpu as pltpu
from jax.experimental.pallas import tpu_sc as plsc

N_NODES = 10000
N_EDGES = 160000
HID = 256
HALF = 128
PE_DIM = 20
NLAYERS = 4
NGRAPH = 64
NCLS = 10

BLK = 2000
NB = N_NODES // BLK

NSUB = 16
EDGES_PER_TILE = N_EDGES // NSUB
CHUNK = 80
NCHUNK = EDGES_PER_TILE // CHUNK
ROWS_PER_TILE = N_NODES // NSUB
ZR = 25


NROW = 4
NIDX = 8


def _seg_body(x2_hbm, src_hbm, dst_hbm, zero_hbm, out_hbm,
              sidx, didx, rows, acc, gsem, ssem, isem):
    c = lax.axis_index("c")
    s = lax.axis_index("s")
    off = c * N_NODES

    pltpu.sync_copy(zero_hbm, acc.at[pl.ds(s * ROWS_PER_TILE, ROWS_PER_TILE)])

    def _idx_load(j, k):
        pltpu.async_copy(src_hbm.at[s, j], sidx.at[k], isem[k])
        pltpu.async_copy(dst_hbm.at[s, j], didx.at[k], isem[k])

    def _idx_wait(k):
        pltpu.make_async_copy(src_hbm.at[0, 0], sidx.at[k], isem[k]).wait()
        pltpu.make_async_copy(src_hbm.at[0, 0], didx.at[k], isem[k]).wait()

    def _add_off(k):
        for v in range(CHUNK // 16):
            sidx[k, pl.ds(v * 16, 16)] = sidx[k, pl.ds(v * 16, 16)] + off

    def _gather(k_idx, k_row):
        pltpu.async_copy(x2_hbm.at[sidx.at[k_idx]], rows.at[k_row], gsem[k_row])

    def _gather_wait(k_row):
        pltpu.make_async_copy(
            x2_hbm.at[pl.ds(0, CHUNK)], rows.at[k_row], gsem[k_row]).wait()

    def _scat(k_idx, k_row):
        pltpu.async_copy(rows.at[k_row], acc.at[didx.at[k_idx]], ssem[k_row],
                         add=True)

    def _scat_drain(k_row):
        pltpu.make_async_copy(
            x2_hbm.at[pl.ds(0, CHUNK)], rows.at[k_row], ssem[k_row]).wait()

    plsc.subcore_barrier()

    _idx_load(0, 0)
    _idx_load(1, 1)
    _idx_wait(0)
    _add_off(0)
    _gather(0, 0)

    @pl.loop(0, NCHUNK)
    def _pipe(j):
        jm8 = j % NIDX
        for m in range(NIDX):
            @pl.when(jm8 == m)
            def _():
                mr = m % NROW
                mn = (m + 1) % NIDX
                mnr = (m + 1) % NROW
                mnn = (m + 2) % NIDX

                @pl.when(j >= 3)
                def _():
                    _scat_drain(mnr)

                @pl.when(j < NCHUNK - 1)
                def _():
                    _idx_wait(mn)
                    _add_off(mn)
                    _gather(mn, mnr)

                @pl.when(j < NCHUNK - 2)
                def _():
                    _idx_load(j + 2, mnn)

                _gather_wait(mr)
                _scat(m, mr)

    for jj in (NCHUNK - 3, NCHUNK - 2, NCHUNK - 1):
        _scat_drain(jj % NROW)

    plsc.subcore_barrier()
    pltpu.sync_copy(
        acc.at[pl.ds(s * ROWS_PER_TILE, ROWS_PER_TILE)],
        out_hbm.at[pl.ds(off + s * ROWS_PER_TILE, ROWS_PER_TILE)],
    )


def _segment_sum_sc(x2, src3, dst3, zslab):
    mesh = plsc.VectorSubcoreMesh(core_axis_name="c", subcore_axis_name="s")
    fn = pl.kernel(
        _seg_body,
        out_type=jax.ShapeDtypeStruct((2 * N_NODES, HALF), jnp.float32),
        mesh=mesh,
        scratch_types=[
            pltpu.VMEM((NIDX, CHUNK), jnp.int32),
            pltpu.VMEM((NIDX, CHUNK), jnp.int32),
            pltpu.VMEM((NROW, CHUNK, HALF), jnp.float32),
            pltpu.VMEM_SHARED((N_NODES, HALF), jnp.float32),
            [pltpu.SemaphoreType.DMA] * NROW,
            [pltpu.SemaphoreType.DMA] * NROW,
            [pltpu.SemaphoreType.DMA] * NIDX,
        ],
        compiler_params=pltpu.CompilerParams(use_tc_tiling_on_sc=False),
    )
    return fn(x2, src3, dst3, zslab)


_INV_N = 1.0 / N_NODES


def _bn_coeffs(st_sum, st_sq, g, b):
    mu = st_sum * _INV_N
    var = st_sq * _INV_N - mu * mu
    sc = g * lax.rsqrt(var + 1e-5)
    sh = b - mu * sc
    return sc, sh


def _accum_stats(st_ref, z, i):
    @pl.when(i == 0)
    def _():
        st_ref[...] = jnp.zeros_like(st_ref)

    s1 = jnp.sum(z, axis=0)
    s2 = jnp.sum(z * z, axis=0)
    st_ref[...] += jnp.concatenate([s1[None, None, :], s2[None, None, :]], axis=1)


def _proj_body(p_ref, w_ref, b_ref, o_ref):
    o_ref[...] = (
        jnp.dot(p_ref[...], w_ref[0], preferred_element_type=jnp.float32)
        + b_ref[0, 0][None, :]
    )


def _proj(pos_enc, wq, bq):
    return pl.pallas_call(
        _proj_body,
        grid=(2, NB),
        in_specs=[
            pl.BlockSpec((BLK, PE_DIM), lambda h, i: (i, 0)),
            pl.BlockSpec((1, PE_DIM, HALF), lambda h, i: (h, 0, 0)),
            pl.BlockSpec((1, 1, HALF), lambda h, i: (h, 0, 0)),
        ],
        out_specs=pl.BlockSpec((BLK, HALF), lambda h, i: (h * NB + i, 0)),
        out_shape=jax.ShapeDtypeStruct((2 * N_NODES, HALF), jnp.float32),
    )(pos_enc, wq, bq)


def _mlp1_body(e_ref, xlo, xhi, nlo, nhi, w_ref, b_ref, z_ref, st_ref):
    i = pl.program_id(1)
    efac = 1.0 + e_ref[0, 0]
    y = jnp.concatenate(
        [efac * xlo[...] + nlo[...], efac * xhi[...] + nhi[...]], axis=1
    )
    z = jnp.dot(y, w_ref[0], preferred_element_type=jnp.float32) + b_ref[0, 0][None, :]
    z_ref[...] = z
    _accum_stats(st_ref, z, i)


def _mlp1(eps_i, x2, n2, wq, bq):
    return pl.pallas_call(
        _mlp1_body,
        grid=(2, NB),
        in_specs=[
            pl.BlockSpec((1, 1), lambda h, i: (0, 0)),
            pl.BlockSpec((BLK, HALF), lambda h, i: (i, 0)),
            pl.BlockSpec((BLK, HALF), lambda h, i: (NB + i, 0)),
            pl.BlockSpec((BLK, HALF), lambda h, i: (i, 0)),
            pl.BlockSpec((BLK, HALF), lambda h, i: (NB + i, 0)),
            pl.BlockSpec((1, HID, HALF), lambda h, i: (h, 0, 0)),
            pl.BlockSpec((1, 1, HALF), lambda h, i: (h, 0, 0)),
        ],
        out_specs=[
            pl.BlockSpec((BLK, HALF), lambda h, i: (h * NB + i, 0)),
            pl.BlockSpec((1, 2, HALF), lambda h, i: (h, 0, 0)),
        ],
        out_shape=[
            jax.ShapeDtypeStruct((2 * N_NODES, HALF), jnp.float32),
            jax.ShapeDtypeStruct((2, 2, HALF), jnp.float32),
        ],
    )(eps_i, x2, x2, n2, n2, wq, bq)


def _mlp2_body(st1, g_ref, bt_ref, w_ref, b_ref, zlo, zhi, z_ref, st_ref):
    i = pl.program_id(1)
    parts = []
    for a, zr in ((0, zlo), (1, zhi)):
        sc, sh = _bn_coeffs(st1[a, 0, :], st1[a, 1, :], g_ref[a], bt_ref[a])
        parts.append(jnp.maximum(zr[...] * sc[None, :] + sh[None, :], 0.0))
    y = jnp.concatenate(parts, axis=1)
    z = jnp.dot(y, w_ref[0], preferred_element_type=jnp.float32) + b_ref[0, 0][None, :]
    z_ref[...] = z
    _accum_stats(st_ref, z, i)


def _mlp2(st1, g, bt, z1, wq, bq):
    return pl.pallas_call(
        _mlp2_body,
        grid=(2, NB),
        in_specs=[
            pl.BlockSpec((2, 2, HALF), lambda h, i: (0, 0, 0)),
            pl.BlockSpec((2, HALF), lambda h, i: (0, 0)),
            pl.BlockSpec((2, HALF), lambda h, i: (0, 0)),
            pl.BlockSpec((1, HID, HALF), lambda h, i: (h, 0, 0)),
            pl.BlockSpec((1, 1, HALF), lambda h, i: (h, 0, 0)),
            pl.BlockSpec((BLK, HALF), lambda h, i: (i, 0)),
            pl.BlockSpec((BLK, HALF), lambda h, i: (NB + i, 0)),
        ],
        out_specs=[
            pl.BlockSpec((BLK, HALF), lambda h, i: (h * NB + i, 0)),
            pl.BlockSpec((1, 2, HALF), lambda h, i: (h, 0, 0)),
        ],
        out_shape=[
            jax.ShapeDtypeStruct((2 * N_NODES, HALF), jnp.float32),
            jax.ShapeDtypeStruct((2, 2, HALF), jnp.float32),
        ],
    )(st1, g, bt, wq, bq, z1, z1)


def _bnrelu_body(st_in, g_ref, b_ref, z_ref, o_ref, st_ref):
    i = pl.program_id(1)
    sc, sh = _bn_coeffs(st_in[0, 0, :], st_in[0, 1, :], g_ref[0, 0], b_ref[0, 0])
    val = jnp.maximum(z_ref[...] * sc[None, :] + sh[None, :], 0.0)
    o_ref[...] = val
    _accum_stats(st_ref, val, i)


def _bnrelu(st_in, g, b, z):
    return pl.pallas_call(
        _bnrelu_body,
        grid=(2, NB),
        in_specs=[
            pl.BlockSpec((1, 2, HALF), lambda h, i: (h, 0, 0)),
            pl.BlockSpec((1, 1, HALF), lambda h, i: (h, 0, 0)),
            pl.BlockSpec((1, 1, HALF), lambda h, i: (h, 0, 0)),
            pl.BlockSpec((BLK, HALF), lambda h, i: (h * NB + i, 0)),
        ],
        out_specs=[
            pl.BlockSpec((BLK, HALF), lambda h, i: (h * NB + i, 0)),
            pl.BlockSpec((1, 2, HALF), lambda h, i: (h, 0, 0)),
        ],
        out_shape=[
            jax.ShapeDtypeStruct((2 * N_NODES, HALF), jnp.float32),
            jax.ShapeDtypeStruct((2, 2, HALF), jnp.float32),
        ],
    )(st_in, g, b, z)


def _bnres_body(st_in, g_ref, b_ref, bf_ref, x_ref, o_ref):
    sc, sh = _bn_coeffs(st_in[0, 0, :], st_in[0, 1, :], g_ref[0, 0], b_ref[0, 0])
    o_ref[...] = x_ref[...] + jnp.maximum(bf_ref[...] * sc[None, :] + sh[None, :], 0.0)


def _bnres(st_in, g, b, bf, x2):
    return pl.pallas_call(
        _bnres_body,
        grid=(2, NB),
        in_specs=[
            pl.BlockSpec((1, 2, HALF), lambda h, i: (h, 0, 0)),
            pl.BlockSpec((1, 1, HALF), lambda h, i: (h, 0, 0)),
            pl.BlockSpec((1, 1, HALF), lambda h, i: (h, 0, 0)),
            pl.BlockSpec((BLK, HALF), lambda h, i: (h * NB + i, 0)),
            pl.BlockSpec((BLK, HALF), lambda h, i: (h * NB + i, 0)),
        ],
        out_specs=pl.BlockSpec((BLK, HALF), lambda h, i: (h * NB + i, 0)),
        out_shape=jax.ShapeDtypeStruct((2 * N_NODES, HALF), jnp.float32),
    )(st_in, g, b, bf, x2)


def _pool_body(gid_ref, wp_ref, bp_ref, *refs):
    o_ref = refs[-1]
    h_refs = refs[:-1]
    i = pl.program_id(0)
    v = jnp.zeros((BLK, HALF), jnp.float32)
    for k in range(NLAYERS + 1):
        v = v + jnp.dot(
            h_refs[2 * k][...], wp_ref[k, 0], preferred_element_type=jnp.float32
        )
        v = v + jnp.dot(
            h_refs[2 * k + 1][...], wp_ref[k, 1], preferred_element_type=jnp.float32
        )
    gid = gid_ref[0, 0, :]
    onehot = (
        lax.broadcasted_iota(jnp.int32, (NGRAPH, BLK), 0) == gid[None, :]
    ).astype(jnp.float32)
    contrib = jnp.dot(onehot, v, preferred_element_type=jnp.float32)

    @pl.when(i == 0)
    def _():
        o_ref[...] = jnp.broadcast_to(bp_ref[0][None, :], (NGRAPH, HALF))

    o_ref[...] += contrib


def _pool(gids3, wp_all, bp_pad, hiddens):
    n_h = NLAYERS + 1
    in_specs = [
        pl.BlockSpec((1, 1, BLK), lambda i: (i, 0, 0)),
        pl.BlockSpec((n_h, 2, HALF, HALF), lambda i: (0, 0, 0, 0)),
        pl.BlockSpec((1, HALF), lambda i: (0, 0)),
    ]
    args = [gids3, wp_all, bp_pad]
    for x2 in hiddens:
        in_specs.append(pl.BlockSpec((BLK, HALF), lambda i: (i, 0)))
        in_specs.append(pl.BlockSpec((BLK, HALF), lambda i: (NB + i, 0)))
        args.append(x2)
        args.append(x2)
    return pl.pallas_call(
        _pool_body,
        grid=(NB,),
        in_specs=in_specs,
        out_specs=pl.BlockSpec((NGRAPH, HALF), lambda i: (0, 0)),
        out_shape=jax.ShapeDtypeStruct((NGRAPH, HALF), jnp.float32),
    )(*args)


def kernel(h, edge_index, e, pos_enc, graph_ids, Wpe, bpe, eps, W1, b1, g1, bt1,
           W2, b2, ga, ba, gl, bl, Wp, bp):
    src3 = edge_index[0].reshape(NSUB, NCHUNK, CHUNK)
    dst3 = edge_index[1].reshape(NSUB, NCHUNK, CHUNK)
    zslab = jnp.zeros((ROWS_PER_TILE, HALF), jnp.float32)

    wpe_q = Wpe.reshape(PE_DIM, 2, HALF).transpose(1, 0, 2)
    bpe_q = bpe.reshape(2, 1, HALF)
    x2 = _proj(pos_enc, wpe_q, bpe_q)

    hiddens = [x2]
    for i in range(NLAYERS):
        w1q = W1[i].reshape(HID, 2, HALF).transpose(1, 0, 2)
        b1q = b1[i].reshape(2, 1, HALF)
        w2q = W2[i].reshape(HID, 2, HALF).transpose(1, 0, 2)
        b2q = b2[i].reshape(2, 1, HALF)
        eps_i = eps[i].reshape(1, 1)

        neigh2 = _segment_sum_sc(x2, src3, dst3, zslab)
        z1, st1 = _mlp1(eps_i, x2, neigh2, w1q, b1q)
        z2, st2 = _mlp2(st1, g1[i].reshape(2, HALF), bt1[i].reshape(2, HALF),
                        z1, w2q, b2q)
        bf, st3 = _bnrelu(st2, ga[i].reshape(2, 1, HALF), ba[i].reshape(2, 1, HALF), z2)
        x2 = _bnres(st3, gl[i].reshape(2, 1, HALF), bl[i].reshape(2, 1, HALF), bf, x2)
        hiddens.append(x2)

    gids3 = graph_ids.reshape(NB, 1, BLK)
    wp_all = jnp.pad(Wp, ((0, 0), (0, 0), (0, HALF - NCLS))).reshape(
        NLAYERS + 1, 2, HALF, HALF
    )
    bp_pad = jnp.pad(jnp.sum(bp, axis=0), (0, HALF - NCLS)).reshape(1, HALF)
    score_pad = _pool(gids3, wp_all, bp_pad, hiddens)
    return score_pad[:, :NCLS]


# speedup vs baseline: 6.5324x; 1.0145x over previous
import functools

import jax
import jax.numpy as jnp
from jax import lax
from jax.experimental import pallas as pl
from jax.experimental.pallas import tpu as pltpu
from jax.experimental.pallas import tpu_sc as plsc

N_NODES = 10000
N_EDGES = 160000
HID = 256
HALF = 128
PE_DIM = 20
NLAYERS = 4
NGRAPH = 64
NCLS = 10

BLK = 2000
NB = N_NODES // BLK

NSUB = 16
EDGES_PER_TILE = N_EDGES // NSUB
CHUNK = 80
NCHUNK = EDGES_PER_TILE // CHUNK
ROWS_PER_TILE = N_NODES // NSUB
ZR = 25


NROW = 4
NIDX = 8


def _seg_body(x2_hbm, src_hbm, dst_hbm, zero_hbm, out_hbm,
              sidx, didx, rows, acc, gsem, ssem, isem):
    c = lax.axis_index("c")
    s = lax.axis_index("s")
    off = c * N_NODES

    pltpu.sync_copy(zero_hbm, acc.at[pl.ds(s * ROWS_PER_TILE, ROWS_PER_TILE)])

    def _idx_load(j, k):
        pltpu.async_copy(src_hbm.at[s, j], sidx.at[k], isem[k])
        pltpu.async_copy(dst_hbm.at[s, j], didx.at[k], isem[k])

    def _idx_wait(k):
        pltpu.make_async_copy(src_hbm.at[0, 0], sidx.at[k], isem[k]).wait()
        pltpu.make_async_copy(src_hbm.at[0, 0], didx.at[k], isem[k]).wait()

    def _add_off(k):
        for v in range(CHUNK // 16):
            sidx[k, pl.ds(v * 16, 16)] = sidx[k, pl.ds(v * 16, 16)] + off

    def _gather(k_idx, k_row):
        pltpu.async_copy(x2_hbm.at[sidx.at[k_idx]], rows.at[k_row], gsem[k_row])

    def _gather_wait(k_row):
        pltpu.make_async_copy(
            x2_hbm.at[pl.ds(0, CHUNK)], rows.at[k_row], gsem[k_row]).wait()

    def _scat(k_idx, k_row):
        pltpu.async_copy(rows.at[k_row], acc.at[didx.at[k_idx]], ssem[k_row],
                         add=True)

    def _scat_drain(k_row):
        pltpu.make_async_copy(
            x2_hbm.at[pl.ds(0, CHUNK)], rows.at[k_row], ssem[k_row]).wait()

    plsc.subcore_barrier()

    _idx_load(0, 0)
    _idx_load(1, 1)
    _idx_wait(0)
    _add_off(0)
    _gather(0, 0)

    @pl.loop(0, NCHUNK)
    def _pipe(j):
        jm8 = j % NIDX
        for m in range(NIDX):
            @pl.when(jm8 == m)
            def _():
                mr = m % NROW
                mn = (m + 1) % NIDX
                mnr = (m + 1) % NROW
                mnn = (m + 2) % NIDX

                @pl.when(j >= 3)
                def _():
                    _scat_drain(mnr)

                @pl.when(j < NCHUNK - 1)
                def _():
                    _idx_wait(mn)
                    _add_off(mn)
                    _gather(mn, mnr)

                @pl.when(j < NCHUNK - 2)
                def _():
                    _idx_load(j + 2, mnn)

                _gather_wait(mr)
                _scat(m, mr)

    for jj in (NCHUNK - 3, NCHUNK - 2, NCHUNK - 1):
        _scat_drain(jj % NROW)

    plsc.subcore_barrier()
    pltpu.sync_copy(
        acc.at[pl.ds(s * ROWS_PER_TILE, ROWS_PER_TILE)],
        out_hbm.at[pl.ds(off + s * ROWS_PER_TILE, ROWS_PER_TILE)],
    )


def _segment_sum_sc(x2, src3, dst3, zslab):
    mesh = plsc.VectorSubcoreMesh(core_axis_name="c", subcore_axis_name="s")
    fn = pl.kernel(
        _seg_body,
        out_type=jax.ShapeDtypeStruct((2 * N_NODES, HALF), jnp.float32),
        mesh=mesh,
        scratch_types=[
            pltpu.VMEM((NIDX, CHUNK), jnp.int32),
            pltpu.VMEM((NIDX, CHUNK), jnp.int32),
            pltpu.VMEM((NROW, CHUNK, HALF), jnp.float32),
            pltpu.VMEM_SHARED((N_NODES, HALF), jnp.float32),
            [pltpu.SemaphoreType.DMA] * NROW,
            [pltpu.SemaphoreType.DMA] * NROW,
            [pltpu.SemaphoreType.DMA] * NIDX,
        ],
        compiler_params=pltpu.CompilerParams(use_tc_tiling_on_sc=False),
    )
    return fn(x2, src3, dst3, zslab)


_INV_N = 1.0 / N_NODES


def _bn_coeffs(st_sum, st_sq, g, b):
    mu = st_sum * _INV_N
    var = st_sq * _INV_N - mu * mu
    sc = g * lax.rsqrt(var + 1e-5)
    sh = b - mu * sc
    return sc, sh


def _accum_stats(st_ref, z, i):
    @pl.when(i == 0)
    def _():
        st_ref[...] = jnp.zeros_like(st_ref)

    s1 = jnp.sum(z, axis=0)
    s2 = jnp.sum(z * z, axis=0)
    st_ref[...] += jnp.concatenate([s1[None, None, :], s2[None, None, :]], axis=1)


def _proj_body(p_ref, w_ref, b_ref, o_ref):
    o_ref[...] = (
        jnp.dot(p_ref[...], w_ref[...], preferred_element_type=jnp.float32)
        + b_ref[0, 0][None, :]
    )


def _proj(pos_enc, wq, bq):
    return pl.pallas_call(
        _proj_body,
        grid=(2, NB),
        in_specs=[
            pl.BlockSpec((BLK, PE_DIM), lambda h, i: (i, 0)),
            pl.BlockSpec((PE_DIM, HALF), lambda h, i: (0, h)),
            pl.BlockSpec((1, 1, HALF), lambda h, i: (h, 0, 0)),
        ],
        out_specs=pl.BlockSpec((BLK, HALF), lambda h, i: (h * NB + i, 0)),
        out_shape=jax.ShapeDtypeStruct((2 * N_NODES, HALF), jnp.float32),
    )(pos_enc, wq, bq)


def _mlp1_body(e_ref, xlo, xhi, nlo, nhi, w_ref, b_ref, z_ref, st_ref):
    i = pl.program_id(1)
    efac = 1.0 + e_ref[0, 0, 0]
    y = jnp.concatenate(
        [efac * xlo[...] + nlo[...], efac * xhi[...] + nhi[...]], axis=1
    )
    z = jnp.dot(y, w_ref[0], preferred_element_type=jnp.float32) + b_ref[0, 0, 0][None, :]
    z_ref[...] = z
    _accum_stats(st_ref, z, i)


def _mlp1(li, eps2, x2, n2, w, b3):
    return pl.pallas_call(
        _mlp1_body,
        grid=(2, NB),
        in_specs=[
            pl.BlockSpec((1, 1, 1), lambda h, i, li=li: (li, 0, 0)),
            pl.BlockSpec((BLK, HALF), lambda h, i: (i, 0)),
            pl.BlockSpec((BLK, HALF), lambda h, i: (NB + i, 0)),
            pl.BlockSpec((BLK, HALF), lambda h, i: (i, 0)),
            pl.BlockSpec((BLK, HALF), lambda h, i: (NB + i, 0)),
            pl.BlockSpec((1, HID, HALF), lambda h, i, li=li: (li, 0, h)),
            pl.BlockSpec((1, 1, 1, HALF), lambda h, i, li=li: (li, h, 0, 0)),
        ],
        out_specs=[
            pl.BlockSpec((BLK, HALF), lambda h, i: (h * NB + i, 0)),
            pl.BlockSpec((1, 2, HALF), lambda h, i: (h, 0, 0)),
        ],
        out_shape=[
            jax.ShapeDtypeStruct((2 * N_NODES, HALF), jnp.float32),
            jax.ShapeDtypeStruct((2, 2, HALF), jnp.float32),
        ],
    )(eps2, x2, x2, n2, n2, w, b3)


def _mlp2_body(st1, g_ref, bt_ref, w_ref, b_ref, zlo, zhi, z_ref, st_ref):
    i = pl.program_id(1)
    parts = []
    for a, zr in ((0, zlo), (1, zhi)):
        sc, sh = _bn_coeffs(st1[a, 0, :], st1[a, 1, :], g_ref[0, a], bt_ref[0, a])
        parts.append(jnp.maximum(zr[...] * sc[None, :] + sh[None, :], 0.0))
    y = jnp.concatenate(parts, axis=1)
    z = jnp.dot(y, w_ref[0], preferred_element_type=jnp.float32) + b_ref[0, 0, 0][None, :]
    z_ref[...] = z
    _accum_stats(st_ref, z, i)


def _mlp2(li, st1, g3, bt3, z1, w, b3):
    return pl.pallas_call(
        _mlp2_body,
        grid=(2, NB),
        in_specs=[
            pl.BlockSpec((2, 2, HALF), lambda h, i: (0, 0, 0)),
            pl.BlockSpec((1, 2, HALF), lambda h, i, li=li: (li, 0, 0)),
            pl.BlockSpec((1, 2, HALF), lambda h, i, li=li: (li, 0, 0)),
            pl.BlockSpec((1, HID, HALF), lambda h, i, li=li: (li, 0, h)),
            pl.BlockSpec((1, 1, 1, HALF), lambda h, i, li=li: (li, h, 0, 0)),
            pl.BlockSpec((BLK, HALF), lambda h, i: (i, 0)),
            pl.BlockSpec((BLK, HALF), lambda h, i: (NB + i, 0)),
        ],
        out_specs=[
            pl.BlockSpec((BLK, HALF), lambda h, i: (h * NB + i, 0)),
            pl.BlockSpec((1, 2, HALF), lambda h, i: (h, 0, 0)),
        ],
        out_shape=[
            jax.ShapeDtypeStruct((2 * N_NODES, HALF), jnp.float32),
            jax.ShapeDtypeStruct((2, 2, HALF), jnp.float32),
        ],
    )(st1, g3, bt3, w, b3, z1, z1)


def _bnstat_body(st_in, g_ref, b_ref, z_ref, st_ref):
    i = pl.program_id(1)
    sc, sh = _bn_coeffs(st_in[0, 0, :], st_in[0, 1, :], g_ref[0, 0, 0], b_ref[0, 0, 0])
    val = jnp.maximum(z_ref[...] * sc[None, :] + sh[None, :], 0.0)
    _accum_stats(st_ref, val, i)


def _bnstat(li, st2, g3, b3, z2):
    return pl.pallas_call(
        _bnstat_body,
        grid=(2, NB),
        in_specs=[
            pl.BlockSpec((1, 2, HALF), lambda h, i: (h, 0, 0)),
            pl.BlockSpec((1, 1, 1, HALF), lambda h, i, li=li: (li, h, 0, 0)),
            pl.BlockSpec((1, 1, 1, HALF), lambda h, i, li=li: (li, h, 0, 0)),
            pl.BlockSpec((BLK, HALF), lambda h, i: (h * NB + i, 0)),
        ],
        out_specs=pl.BlockSpec((1, 2, HALF), lambda h, i: (h, 0, 0)),
        out_shape=jax.ShapeDtypeStruct((2, 2, HALF), jnp.float32),
    )(st2, g3, b3, z2)


def _bnres_body(st2, st3, g2_ref, b2_ref, g3_ref, b3_ref, z_ref, x_ref, o_ref):
    sc2, sh2 = _bn_coeffs(st2[0, 0, :], st2[0, 1, :], g2_ref[0, 0, 0], b2_ref[0, 0, 0])
    sc3, sh3 = _bn_coeffs(st3[0, 0, :], st3[0, 1, :], g3_ref[0, 0, 0], b3_ref[0, 0, 0])
    val = jnp.maximum(z_ref[...] * sc2[None, :] + sh2[None, :], 0.0)
    o_ref[...] = x_ref[...] + jnp.maximum(val * sc3[None, :] + sh3[None, :], 0.0)


def _bnres(li, st2, st3, g2, b2, g3, b3, z2, x2):
    return pl.pallas_call(
        _bnres_body,
        grid=(2, NB),
        in_specs=[
            pl.BlockSpec((1, 2, HALF), lambda h, i: (h, 0, 0)),
            pl.BlockSpec((1, 2, HALF), lambda h, i: (h, 0, 0)),
            pl.BlockSpec((1, 1, 1, HALF), lambda h, i, li=li: (li, h, 0, 0)),
            pl.BlockSpec((1, 1, 1, HALF), lambda h, i, li=li: (li, h, 0, 0)),
            pl.BlockSpec((1, 1, 1, HALF), lambda h, i, li=li: (li, h, 0, 0)),
            pl.BlockSpec((1, 1, 1, HALF), lambda h, i, li=li: (li, h, 0, 0)),
            pl.BlockSpec((BLK, HALF), lambda h, i: (h * NB + i, 0)),
            pl.BlockSpec((BLK, HALF), lambda h, i: (h * NB + i, 0)),
        ],
        out_specs=pl.BlockSpec((BLK, HALF), lambda h, i: (h * NB + i, 0)),
        out_shape=jax.ShapeDtypeStruct((2 * N_NODES, HALF), jnp.float32),
    )(st2, st3, g2, b2, g3, b3, z2, x2)


def _pool_body(gid_ref, wp_ref, bp_ref, *refs):
    o_ref = refs[-1]
    h_refs = refs[:-1]
    i = pl.program_id(0)
    v = jnp.zeros((BLK, HALF), jnp.float32)
    for k in range(NLAYERS + 1):
        v = v + jnp.dot(
            h_refs[2 * k][...], wp_ref[k, 0], preferred_element_type=jnp.float32
        )
        v = v + jnp.dot(
            h_refs[2 * k + 1][...], wp_ref[k, 1], preferred_element_type=jnp.float32
        )
    gid = gid_ref[0, 0, :]
    onehot = (
        lax.broadcasted_iota(jnp.int32, (NGRAPH, BLK), 0) == gid[None, :]
    ).astype(jnp.float32)
    contrib = jnp.dot(onehot, v, preferred_element_type=jnp.float32)

    @pl.when(i == 0)
    def _():
        o_ref[...] = jnp.broadcast_to(bp_ref[0][None, :], (NGRAPH, HALF))

    o_ref[...] += contrib


def _pool(gids3, wp_all, bp_pad, hiddens):
    n_h = NLAYERS + 1
    in_specs = [
        pl.BlockSpec((1, 1, BLK), lambda i: (i, 0, 0)),
        pl.BlockSpec((n_h, 2, HALF, HALF), lambda i: (0, 0, 0, 0)),
        pl.BlockSpec((1, HALF), lambda i: (0, 0)),
    ]
    args = [gids3, wp_all, bp_pad]
    for x2 in hiddens:
        in_specs.append(pl.BlockSpec((BLK, HALF), lambda i: (i, 0)))
        in_specs.append(pl.BlockSpec((BLK, HALF), lambda i: (NB + i, 0)))
        args.append(x2)
        args.append(x2)
    return pl.pallas_call(
        _pool_body,
        grid=(NB,),
        in_specs=in_specs,
        out_specs=pl.BlockSpec((NGRAPH, HALF), lambda i: (0, 0)),
        out_shape=jax.ShapeDtypeStruct((NGRAPH, HALF), jnp.float32),
    )(*args)


def kernel(h, edge_index, e, pos_enc, graph_ids, Wpe, bpe, eps, W1, b1, g1, bt1,
           W2, b2, ga, ba, gl, bl, Wp, bp):
    src3 = edge_index[0].reshape(NSUB, NCHUNK, CHUNK)
    dst3 = edge_index[1].reshape(NSUB, NCHUNK, CHUNK)
    zslab = jnp.zeros((ROWS_PER_TILE, HALF), jnp.float32)

    x2 = _proj(pos_enc, Wpe, bpe.reshape(2, 1, HALF))

    eps2 = eps.reshape(NLAYERS, 1, 1)
    b1_3 = b1.reshape(NLAYERS, 2, 1, HALF)
    b2_3 = b2.reshape(NLAYERS, 2, 1, HALF)
    g1_3 = g1.reshape(NLAYERS, 2, HALF)
    bt1_3 = bt1.reshape(NLAYERS, 2, HALF)
    ga_3 = ga.reshape(NLAYERS, 2, 1, HALF)
    ba_3 = ba.reshape(NLAYERS, 2, 1, HALF)
    gl_3 = gl.reshape(NLAYERS, 2, 1, HALF)
    bl_3 = bl.reshape(NLAYERS, 2, 1, HALF)

    hiddens = [x2]
    for li in range(NLAYERS):
        neigh2 = _segment_sum_sc(x2, src3, dst3, zslab)
        z1, st1 = _mlp1(li, eps2, x2, neigh2, W1, b1_3)
        z2, st2 = _mlp2(li, st1, g1_3, bt1_3, z1, W2, b2_3)
        st3 = _bnstat(li, st2, ga_3, ba_3, z2)
        x2 = _bnres(li, st2, st3, ga_3, ba_3, gl_3, bl_3, z2, x2)
        hiddens.append(x2)

    gids3 = graph_ids.reshape(NB, 1, BLK)
    wp_all = jnp.pad(Wp, ((0, 0), (0, 0), (0, HALF - NCLS))).reshape(
        NLAYERS + 1, 2, HALF, HALF
    )
    bp_pad = jnp.pad(jnp.sum(bp, axis=0), (0, HALF - NCLS)).reshape(1, HALF)
    score_pad = _pool(gids3, wp_all, bp_pad, hiddens)
    return score_pad[:, :NCLS]


# speedup vs baseline: 7.1708x; 1.0977x over previous
import functools

import jax
import jax.numpy as jnp
from jax import lax
from jax.experimental import pallas as pl
from jax.experimental.pallas import tpu as pltpu
from jax.experimental.pallas import tpu_sc as plsc

N_NODES = 10000
N_EDGES = 160000
HID = 256
HALF = 128
PE_DIM = 20
NLAYERS = 4
NGRAPH = 64
NCLS = 10

BLK = 2000
NB = N_NODES // BLK

NSUB = 16
EDGES_PER_TILE = N_EDGES // NSUB
CHUNK = 80
NCHUNK = EDGES_PER_TILE // CHUNK
ROWS_PER_TILE = N_NODES // NSUB
ZR = 25


NROW = 4
NIDX = 8


def _seg_body(x2_hbm, src_hbm, dst_hbm, zero_hbm, out_hbm,
              sidx, didx, rows, acc, gsem, ssem, isem):
    c = lax.axis_index("c")
    s = lax.axis_index("s")
    off = c * N_NODES

    pltpu.sync_copy(zero_hbm, acc.at[pl.ds(s * ROWS_PER_TILE, ROWS_PER_TILE)])

    def _idx_load(j, k):
        pltpu.async_copy(src_hbm.at[s, j], sidx.at[k], isem[k])
        pltpu.async_copy(dst_hbm.at[s, j], didx.at[k], isem[k])

    def _idx_wait(k):
        pltpu.make_async_copy(src_hbm.at[0, 0], sidx.at[k], isem[k]).wait()
        pltpu.make_async_copy(src_hbm.at[0, 0], didx.at[k], isem[k]).wait()

    def _add_off(k):
        for v in range(CHUNK // 16):
            sidx[k, pl.ds(v * 16, 16)] = sidx[k, pl.ds(v * 16, 16)] + off

    def _gather(k_idx, k_row):
        pltpu.async_copy(x2_hbm.at[sidx.at[k_idx]], rows.at[k_row], gsem[k_row])

    def _gather_wait(k_row):
        pltpu.make_async_copy(
            x2_hbm.at[pl.ds(0, CHUNK)], rows.at[k_row], gsem[k_row]).wait()

    def _scat(k_idx, k_row):
        pltpu.async_copy(rows.at[k_row], acc.at[didx.at[k_idx]], ssem[k_row],
                         add=True)

    def _scat_drain(k_row):
        pltpu.make_async_copy(
            x2_hbm.at[pl.ds(0, CHUNK)], rows.at[k_row], ssem[k_row]).wait()

    plsc.subcore_barrier()

    _idx_load(0, 0)
    _idx_load(1, 1)
    _idx_wait(0)
    _add_off(0)
    _gather(0, 0)

    @pl.loop(0, NCHUNK)
    def _pipe(j):
        jm8 = j % NIDX
        for m in range(NIDX):
            @pl.when(jm8 == m)
            def _():
                mr = m % NROW
                mn = (m + 1) % NIDX
                mnr = (m + 1) % NROW
                mnn = (m + 2) % NIDX

                @pl.when(j >= 3)
                def _():
                    _scat_drain(mnr)

                @pl.when(j < NCHUNK - 1)
                def _():
                    _idx_wait(mn)
                    _add_off(mn)
                    _gather(mn, mnr)

                @pl.when(j < NCHUNK - 2)
                def _():
                    _idx_load(j + 2, mnn)

                _gather_wait(mr)
                _scat(m, mr)

    for jj in (NCHUNK - 3, NCHUNK - 2, NCHUNK - 1):
        _scat_drain(jj % NROW)

    plsc.subcore_barrier()
    pltpu.sync_copy(
        acc.at[pl.ds(s * ROWS_PER_TILE, ROWS_PER_TILE)],
        out_hbm.at[pl.ds(off + s * ROWS_PER_TILE, ROWS_PER_TILE)],
    )


def _segment_sum_sc(x2, src3, dst3, zslab):
    mesh = plsc.VectorSubcoreMesh(core_axis_name="c", subcore_axis_name="s")
    fn = pl.kernel(
        _seg_body,
        out_type=jax.ShapeDtypeStruct((2 * N_NODES, HALF), jnp.float32),
        mesh=mesh,
        scratch_types=[
            pltpu.VMEM((NIDX, CHUNK), jnp.int32),
            pltpu.VMEM((NIDX, CHUNK), jnp.int32),
            pltpu.VMEM((NROW, CHUNK, HALF), jnp.float32),
            pltpu.VMEM_SHARED((N_NODES, HALF), jnp.float32),
            [pltpu.SemaphoreType.DMA] * NROW,
            [pltpu.SemaphoreType.DMA] * NROW,
            [pltpu.SemaphoreType.DMA] * NIDX,
        ],
        compiler_params=pltpu.CompilerParams(use_tc_tiling_on_sc=False),
    )
    return fn(x2, src3, dst3, zslab)


_INV_N = 1.0 / N_NODES


def _bn_coeffs(st_ref, g, b):
    mu = st_ref[0] * _INV_N
    var = st_ref[1] * _INV_N - mu * mu
    sc = g * lax.rsqrt(var + 1e-5)
    sh = b - mu * sc
    return sc, sh


def _accum_stats(st_ref, z, i):
    @pl.when(i == 0)
    def _():
        st_ref[...] = jnp.zeros_like(st_ref)

    s1 = jnp.sum(z, axis=0)
    s2 = jnp.sum(z * z, axis=0)
    st_ref[...] += jnp.concatenate([s1[None, :], s2[None, :]], axis=0)


def _store_halves(o_ref, z):
    o_ref[0] = z[:, :HALF]
    o_ref[1] = z[:, HALF:]


def _proj_body(p_ref, w_ref, b_ref, o_ref):
    z = (
        jnp.dot(p_ref[...], w_ref[...], preferred_element_type=jnp.float32)
        + b_ref[0][None, :]
    )
    _store_halves(o_ref, z)


def _proj(pos_enc, w, b):
    return pl.pallas_call(
        _proj_body,
        grid=(NB,),
        in_specs=[
            pl.BlockSpec((BLK, PE_DIM), lambda i: (i, 0)),
            pl.BlockSpec((PE_DIM, HID), lambda i: (0, 0)),
            pl.BlockSpec((1, HID), lambda i: (0, 0)),
        ],
        out_specs=pl.BlockSpec((2, BLK, HALF), lambda i: (0, i, 0)),
        out_shape=jax.ShapeDtypeStruct((2, N_NODES, HALF), jnp.float32),
    )(pos_enc, w, b)


def _mlp1_body(e_ref, x_ref, n_ref, w_ref, b_ref, z_ref, st_ref):
    i = pl.program_id(0)
    efac = 1.0 + e_ref[0, 0, 0]
    y = jnp.concatenate(
        [efac * x_ref[0] + n_ref[0], efac * x_ref[1] + n_ref[1]], axis=1
    )
    z = jnp.dot(y, w_ref[0], preferred_element_type=jnp.float32) + b_ref[0, 0][None, :]
    _store_halves(z_ref, z)
    _accum_stats(st_ref, z, i)


def _mlp1(li, eps3, x3, n3, w, b):
    return pl.pallas_call(
        _mlp1_body,
        grid=(NB,),
        in_specs=[
            pl.BlockSpec((1, 1, 1), lambda i, li=li: (li, 0, 0)),
            pl.BlockSpec((2, BLK, HALF), lambda i: (0, i, 0)),
            pl.BlockSpec((2, BLK, HALF), lambda i: (0, i, 0)),
            pl.BlockSpec((1, HID, HID), lambda i, li=li: (li, 0, 0)),
            pl.BlockSpec((1, 1, HID), lambda i, li=li: (li, 0, 0)),
        ],
        out_specs=[
            pl.BlockSpec((2, BLK, HALF), lambda i: (0, i, 0)),
            pl.BlockSpec((2, HID), lambda i: (0, 0)),
        ],
        out_shape=[
            jax.ShapeDtypeStruct((2, N_NODES, HALF), jnp.float32),
            jax.ShapeDtypeStruct((2, HID), jnp.float32),
        ],
    )(eps3, x3, n3, w, b)


def _mlp2_body(st1, g_ref, bt_ref, w_ref, b_ref, z1_ref, z_ref, st_ref):
    i = pl.program_id(0)
    sc, sh = _bn_coeffs(st1, g_ref[0, 0], bt_ref[0, 0])
    z1 = jnp.concatenate([z1_ref[0], z1_ref[1]], axis=1)
    y = jnp.maximum(z1 * sc[None, :] + sh[None, :], 0.0)
    z = jnp.dot(y, w_ref[0], preferred_element_type=jnp.float32) + b_ref[0, 0][None, :]
    _store_halves(z_ref, z)
    _accum_stats(st_ref, z, i)


def _mlp2(li, st1, g, bt, z1, w, b):
    return pl.pallas_call(
        _mlp2_body,
        grid=(NB,),
        in_specs=[
            pl.BlockSpec((2, HID), lambda i: (0, 0)),
            pl.BlockSpec((1, 1, HID), lambda i, li=li: (li, 0, 0)),
            pl.BlockSpec((1, 1, HID), lambda i, li=li: (li, 0, 0)),
            pl.BlockSpec((1, HID, HID), lambda i, li=li: (li, 0, 0)),
            pl.BlockSpec((1, 1, HID), lambda i, li=li: (li, 0, 0)),
            pl.BlockSpec((2, BLK, HALF), lambda i: (0, i, 0)),
        ],
        out_specs=[
            pl.BlockSpec((2, BLK, HALF), lambda i: (0, i, 0)),
            pl.BlockSpec((2, HID), lambda i: (0, 0)),
        ],
        out_shape=[
            jax.ShapeDtypeStruct((2, N_NODES, HALF), jnp.float32),
            jax.ShapeDtypeStruct((2, HID), jnp.float32),
        ],
    )(st1, g, bt, w, b, z1)


def _bnstat_body(st2, g_ref, b_ref, z_ref, st_ref):
    i = pl.program_id(0)
    sc, sh = _bn_coeffs(st2, g_ref[0, 0], b_ref[0, 0])
    z2 = jnp.concatenate([z_ref[0], z_ref[1]], axis=1)
    val = jnp.maximum(z2 * sc[None, :] + sh[None, :], 0.0)
    _accum_stats(st_ref, val, i)


def _bnstat(li, st2, g, b, z2):
    return pl.pallas_call(
        _bnstat_body,
        grid=(NB,),
        in_specs=[
            pl.BlockSpec((2, HID), lambda i: (0, 0)),
            pl.BlockSpec((1, 1, HID), lambda i, li=li: (li, 0, 0)),
            pl.BlockSpec((1, 1, HID), lambda i, li=li: (li, 0, 0)),
            pl.BlockSpec((2, BLK, HALF), lambda i: (0, i, 0)),
        ],
        out_specs=pl.BlockSpec((2, HID), lambda i: (0, 0)),
        out_shape=jax.ShapeDtypeStruct((2, HID), jnp.float32),
    )(st2, g, b, z2)


def _bnres_body(st2, st3, g2_ref, b2_ref, g3_ref, b3_ref, z_ref, x_ref, o_ref):
    sc2, sh2 = _bn_coeffs(st2, g2_ref[0, 0], b2_ref[0, 0])
    sc3, sh3 = _bn_coeffs(st3, g3_ref[0, 0], b3_ref[0, 0])
    z2 = jnp.concatenate([z_ref[0], z_ref[1]], axis=1)
    x = jnp.concatenate([x_ref[0], x_ref[1]], axis=1)
    val = jnp.maximum(z2 * sc2[None, :] + sh2[None, :], 0.0)
    out = x + jnp.maximum(val * sc3[None, :] + sh3[None, :], 0.0)
    _store_halves(o_ref, out)


def _bnres(li, st2, st3, g2, b2, g3, b3, z2, x3):
    return pl.pallas_call(
        _bnres_body,
        grid=(NB,),
        in_specs=[
            pl.BlockSpec((2, HID), lambda i: (0, 0)),
            pl.BlockSpec((2, HID), lambda i: (0, 0)),
            pl.BlockSpec((1, 1, HID), lambda i, li=li: (li, 0, 0)),
            pl.BlockSpec((1, 1, HID), lambda i, li=li: (li, 0, 0)),
            pl.BlockSpec((1, 1, HID), lambda i, li=li: (li, 0, 0)),
            pl.BlockSpec((1, 1, HID), lambda i, li=li: (li, 0, 0)),
            pl.BlockSpec((2, BLK, HALF), lambda i: (0, i, 0)),
            pl.BlockSpec((2, BLK, HALF), lambda i: (0, i, 0)),
        ],
        out_specs=pl.BlockSpec((2, BLK, HALF), lambda i: (0, i, 0)),
        out_shape=jax.ShapeDtypeStruct((2, N_NODES, HALF), jnp.float32),
    )(st2, st3, g2, b2, g3, b3, z2, x3)


def _pool_body(gid_ref, wp_ref, bp_ref, *refs):
    o_ref = refs[-1]
    h_refs = refs[:-1]
    i = pl.program_id(0)
    v = jnp.zeros((BLK, HALF), jnp.float32)
    for k in range(NLAYERS + 1):
        hk = jnp.concatenate([h_refs[k][0], h_refs[k][1]], axis=1)
        v = v + jnp.dot(hk, wp_ref[k], preferred_element_type=jnp.float32)
    gid = gid_ref[0, 0, :]
    onehot = (
        lax.broadcasted_iota(jnp.int32, (NGRAPH, BLK), 0) == gid[None, :]
    ).astype(jnp.float32)
    contrib = jnp.dot(onehot, v, preferred_element_type=jnp.float32)

    @pl.when(i == 0)
    def _():
        o_ref[...] = jnp.broadcast_to(bp_ref[0][None, :], (NGRAPH, HALF))

    o_ref[...] += contrib


def _pool(gids3, wp_all, bp_pad, hiddens):
    n_h = NLAYERS + 1
    in_specs = [
        pl.BlockSpec((1, 1, BLK), lambda i: (i, 0, 0)),
        pl.BlockSpec((n_h, HID, HALF), lambda i: (0, 0, 0)),
        pl.BlockSpec((1, HALF), lambda i: (0, 0)),
    ]
    args = [gids3, wp_all, bp_pad]
    for x3 in hiddens:
        in_specs.append(pl.BlockSpec((2, BLK, HALF), lambda i: (0, i, 0)))
        args.append(x3)
    return pl.pallas_call(
        _pool_body,
        grid=(NB,),
        in_specs=in_specs,
        out_specs=pl.BlockSpec((NGRAPH, HALF), lambda i: (0, 0)),
        out_shape=jax.ShapeDtypeStruct((NGRAPH, HALF), jnp.float32),
    )(*args)


def kernel(h, edge_index, e, pos_enc, graph_ids, Wpe, bpe, eps, W1, b1, g1, bt1,
           W2, b2, ga, ba, gl, bl, Wp, bp):
    src3 = edge_index[0].reshape(NSUB, NCHUNK, CHUNK)
    dst3 = edge_index[1].reshape(NSUB, NCHUNK, CHUNK)
    zslab = jnp.zeros((ROWS_PER_TILE, HALF), jnp.float32)

    x3 = _proj(pos_enc, Wpe, bpe.reshape(1, HID))

    eps3 = eps.reshape(NLAYERS, 1, 1)
    b1r = b1.reshape(NLAYERS, 1, HID)
    b2r = b2.reshape(NLAYERS, 1, HID)
    g1r = g1.reshape(NLAYERS, 1, HID)
    bt1r = bt1.reshape(NLAYERS, 1, HID)
    gar = ga.reshape(NLAYERS, 1, HID)
    bar = ba.reshape(NLAYERS, 1, HID)
    glr = gl.reshape(NLAYERS, 1, HID)
    blr = bl.reshape(NLAYERS, 1, HID)

    hiddens = [x3]
    for li in range(NLAYERS):
        neigh2 = _segment_sum_sc(x3.reshape(2 * N_NODES, HALF), src3, dst3, zslab)
        n3 = neigh2.reshape(2, N_NODES, HALF)
        z1, st1 = _mlp1(li, eps3, x3, n3, W1, b1r)
        z2, st2 = _mlp2(li, st1, g1r, bt1r, z1, W2, b2r)
        st3 = _bnstat(li, st2, gar, bar, z2)
        x3 = _bnres(li, st2, st3, gar, bar, glr, blr, z2, x3)
        hiddens.append(x3)

    gids3 = graph_ids.reshape(NB, 1, BLK)
    wp_all = jnp.pad(Wp, ((0, 0), (0, 0), (0, HALF - NCLS)))
    bp_pad = jnp.pad(jnp.sum(bp, axis=0), (0, HALF - NCLS)).reshape(1, HALF)
    score_pad = _pool(gids3, wp_all, bp_pad, hiddens)
    return score_pad[:, :NCLS]
